# baseline (speedup 1.0000x reference)
import functools

import jax
import jax.numpy as jnp
from jax import lax
from jax.experimental import pallas as pl
from jax.experimental.pallas import tpu as pltpu
from jax.experimental.pallas import tpu_sc as plsc

B = 16
MAX_LEN = 2048
D = 512
TOTAL = 16384

L = 16
T = 32
NBUF = 4
NW = 32
NTILES = (B * MAX_LEN) // (T * NW)
NGROUPS = NTILES // NBUF
TPS = MAX_LEN // T


def _zero_rows(ref, lo, hi):
    def body(j, carry):
        for c in range(D // L):
            ref[j, pl.ds(c * L, L)] = jnp.zeros((L,), jnp.float32)
        return carry
    lax.fori_loop(lo, hi, body, 0)


@functools.partial(
    pl.kernel,
    out_type=jax.ShapeDtypeStruct((B, MAX_LEN, D), jnp.float32),
    mesh=plsc.VectorSubcoreMesh(core_axis_name="c", subcore_axis_name="s"),
    scratch_types=[
        pltpu.VMEM((32,), jnp.int32),
        pltpu.VMEM((NBUF, T), jnp.int32),
        pltpu.VMEM((T, D), jnp.float32),
        pltpu.VMEM((T, D), jnp.float32),
        pltpu.VMEM((T, D), jnp.float32),
        pltpu.VMEM((T, D), jnp.float32),
        pltpu.VMEM((T, D), jnp.float32),
        pltpu.SemaphoreType.DMA,
        pltpu.SemaphoreType.DMA,
        pltpu.SemaphoreType.DMA,
        pltpu.SemaphoreType.DMA,
        pltpu.SemaphoreType.DMA,
        pltpu.SemaphoreType.DMA,
        pltpu.SemaphoreType.DMA,
        pltpu.SemaphoreType.DMA,
        pltpu.SemaphoreType.DMA,
    ],
)
def _split_sc(cu_hbm, flat_hbm, out_hbm, cu_v, idx_v,
              buf0, buf1, buf2, buf3, zbuf,
              gsem0, gsem1, gsem2, gsem3,
              wsem0, wsem1, wsem2, wsem3, zsem):
    bufs = (buf0, buf1, buf2, buf3)
    gsems = (gsem0, gsem1, gsem2, gsem3)
    wsems = (wsem0, wsem1, wsem2, wsem3)

    wid = lax.axis_index("s") * 2 + lax.axis_index("c")

    pltpu.sync_copy(cu_hbm.at[pl.ds(0, 16)], cu_v.at[pl.ds(0, 16)])
    cu_v[pl.ds(16, L)] = jnp.full((L,), TOTAL, jnp.int32)

    def tile_info(k):
        gt = wid + NW * k
        sg = jnp.minimum(gt // TPS, B)
        row = (gt % TPS) * T
        cu_pair = cu_v[pl.ds(sg, L)]
        cu_b = cu_pair[0]
        seg_len = jnp.minimum(cu_pair[1] - cu_b, MAX_LEN)
        v = jnp.clip(seg_len - row, 0, T)
        return sg, row, cu_b + row, v

    def build_idx(i, src):
        for c in range(T // L):
            lane = src + c * L + lax.iota(jnp.int32, L)
            idx_v[i, pl.ds(c * L, L)] = jnp.minimum(lane, TOTAL - 1)

    def gather(i):
        pltpu.async_copy(flat_hbm.at[idx_v.at[i]], bufs[i], gsems[i])

    def gather_wait(i):
        pltpu.make_async_copy(flat_hbm.at[idx_v.at[i]], bufs[i], gsems[i]).wait()

    def write_start(i, sg, row):
        pltpu.async_copy(bufs[i], out_hbm.at[sg, pl.ds(row, T)], wsems[i])

    def write_wait(i, sg, row):
        pltpu.make_async_copy(bufs[i], out_hbm.at[sg, pl.ds(row, T)],
                              wsems[i]).wait()

    for j in range(NBUF - 1):
        _, _, src, v = tile_info(j)

        @pl.when(v > 0)
        def _(j=j, src=src):
            build_idx(j, src)
            gather(j)

    _zero_rows(zbuf, 0, T)

    def group(g, carry):
        for i in range(NBUF):
            k = g * NBUF + i
            sg, row, _, v = tile_info(k)

            @pl.when(v > 0)
            def _(i=i, sg=sg, row=row, v=v):
                gather_wait(i)

                @pl.when(v < T)
                def _():
                    _zero_rows(bufs[i], v, T)

                write_start(i, sg, row)

            @pl.when(v == 0)
            def _(sg=sg, row=row):
                pltpu.async_copy(zbuf, out_hbm.at[sg, pl.ds(row, T)], zsem)

            j = k + NBUF - 1
            jj = (i + NBUF - 1) % NBUF
            sgj, rowj, srcj, vj = tile_info(j)
            sgp, rowp, _, vp = tile_info(jnp.maximum(j - NBUF, 0))

            @pl.when(j < NTILES)
            def _(jj=jj, j=j, sgj=sgj, rowj=rowj, srcj=srcj, vj=vj,
                  sgp=sgp, rowp=rowp, vp=vp):
                @pl.when((j >= NBUF) & (vp > 0))
                def _():
                    write_wait(jj, sgp, rowp)

                @pl.when(vj > 0)
                def _():
                    build_idx(jj, srcj)
                    gather(jj)
        return carry

    lax.fori_loop(0, NGROUPS, group, 0)

    for k in range(NTILES - NBUF, NTILES):
        sg, row, _, v = tile_info(k)

        @pl.when(v > 0)
        def _(k=k, sg=sg, row=row):
            write_wait(k % NBUF, sg, row)

    def zdrain(g, carry):
        for i in range(NBUF):
            k = g * NBUF + i
            sg, row, _, v = tile_info(k)

            @pl.when(v == 0)
            def _(sg=sg, row=row):
                pltpu.make_async_copy(
                    zbuf, out_hbm.at[sg, pl.ds(row, T)], zsem).wait()
        return carry

    lax.fori_loop(0, NGROUPS, zdrain, 0)


def kernel(flat, cu_seqlens):
    return _split_sc(cu_seqlens, flat)


# speedup vs baseline: 1.0151x; 1.0151x over previous
import functools

import jax
import jax.numpy as jnp
from jax import lax
from jax.experimental import pallas as pl
from jax.experimental.pallas import tpu as pltpu
from jax.experimental.pallas import tpu_sc as plsc

B = 16
MAX_LEN = 2048
D = 512
TOTAL = 16384

L = 16
T = 32
NBUF = 4
NW = 32
ROWS_PER_W = (B * MAX_LEN) // NW
NTILES = ROWS_PER_W // T
NGROUPS = NTILES // NBUF
WPS = MAX_LEN // ROWS_PER_W


def _zero_rows(ref, lo, hi):
    def body(j, carry):
        for c in range(D // L):
            ref[j, pl.ds(c * L, L)] = jnp.zeros((L,), jnp.float32)
        return carry
    lax.fori_loop(lo, hi, body, 0)


@functools.partial(
    pl.kernel,
    out_type=jax.ShapeDtypeStruct((B, MAX_LEN, D), jnp.float32),
    mesh=plsc.VectorSubcoreMesh(core_axis_name="c", subcore_axis_name="s"),
    scratch_types=[
        pltpu.VMEM((32,), jnp.int32),
        pltpu.VMEM((NBUF, T), jnp.int32),
        pltpu.VMEM((T, D), jnp.float32),
        pltpu.VMEM((T, D), jnp.float32),
        pltpu.VMEM((T, D), jnp.float32),
        pltpu.VMEM((T, D), jnp.float32),
        pltpu.VMEM((T, D), jnp.float32),
        pltpu.SemaphoreType.DMA,
        pltpu.SemaphoreType.DMA,
        pltpu.SemaphoreType.DMA,
        pltpu.SemaphoreType.DMA,
        pltpu.SemaphoreType.DMA,
        pltpu.SemaphoreType.DMA,
        pltpu.SemaphoreType.DMA,
        pltpu.SemaphoreType.DMA,
        pltpu.SemaphoreType.DMA,
    ],
)
def _split_sc(cu_hbm, flat_hbm, out_hbm, cu_v, idx_v,
              buf0, buf1, buf2, buf3, zbuf,
              gsem0, gsem1, gsem2, gsem3,
              wsem0, wsem1, wsem2, wsem3, zsem):
    bufs = (buf0, buf1, buf2, buf3)
    gsems = (gsem0, gsem1, gsem2, gsem3)
    wsems = (wsem0, wsem1, wsem2, wsem3)

    wid = lax.axis_index("s") * 2 + lax.axis_index("c")
    seg = wid // WPS
    r0 = (wid % WPS) * ROWS_PER_W

    pltpu.sync_copy(cu_hbm.at[pl.ds(0, 16)], cu_v.at[pl.ds(0, 16)])
    cu_v[pl.ds(16, L)] = jnp.full((L,), TOTAL, jnp.int32)
    cu_pair = cu_v[pl.ds(seg, L)]
    cu_b = cu_pair[0]
    cu_b1 = cu_pair[1]
    seg_len = jnp.minimum(cu_b1 - cu_b, MAX_LEN)
    valid = jnp.clip(seg_len - r0, 0, ROWS_PER_W)
    nfull = valid // T
    rem = valid % T
    nvalid = nfull + jnp.where(rem > 0, 1, 0)

    src0 = cu_b + r0

    def build_idx(i, j):
        for c in range(T // L):
            lane = src0 + j * T + c * L + lax.iota(jnp.int32, L)
            idx_v[i, pl.ds(c * L, L)] = jnp.minimum(lane, TOTAL - 1)

    def gather(i):
        pltpu.async_copy(flat_hbm.at[idx_v.at[i]], bufs[i], gsems[i])

    def gather_wait(i):
        pltpu.make_async_copy(flat_hbm.at[idx_v.at[i]], bufs[i], gsems[i]).wait()

    def write_start(i, j):
        pltpu.async_copy(bufs[i], out_hbm.at[seg, pl.ds(r0 + j * T, T)],
                         wsems[i])

    def write_wait(i, j):
        pltpu.make_async_copy(bufs[i], out_hbm.at[seg, pl.ds(r0 + j * T, T)],
                              wsems[i]).wait()

    for j in range(NBUF - 1):
        @pl.when(j < nvalid)
        def _(j=j):
            build_idx(j, j)
            gather(j)

    _zero_rows(zbuf, 0, T)

    def group(g, carry):
        for i in range(NBUF):
            k = g * NBUF + i

            @pl.when(k < nvalid)
            def _(k=k, i=i):
                gather_wait(i)

                @pl.when((k == nfull) & (rem > 0))
                def _():
                    _zero_rows(bufs[i], rem, T)

                write_start(i, k)

                j = k + NBUF - 1
                jj = (i + NBUF - 1) % NBUF

                @pl.when(j < nvalid)
                def _(j=j, jj=jj):
                    @pl.when(j >= NBUF)
                    def _():
                        write_wait(jj, j - NBUF)
                    build_idx(jj, j)
                    gather(jj)

            @pl.when(k >= nvalid)
            def _(k=k):
                pltpu.async_copy(
                    zbuf, out_hbm.at[seg, pl.ds(r0 + k * T, T)], zsem)
        return carry

    lax.fori_loop(0, NGROUPS, group, 0)

    def drain(g, carry):
        for i in range(NBUF):
            k = g * NBUF + i

            @pl.when((k < nvalid) & (k + NBUF >= nvalid))
            def _(k=k, i=i):
                write_wait(i, k)

            @pl.when(k >= nvalid)
            def _(k=k):
                pltpu.make_async_copy(
                    zbuf, out_hbm.at[seg, pl.ds(r0 + k * T, T)], zsem).wait()
        return carry

    lax.fori_loop(0, NGROUPS, drain, 0)


def kernel(flat, cu_seqlens):
    return _split_sc(cu_seqlens, flat)
